# baseline (speedup 1.0000x reference)
import functools

import jax
import jax.numpy as jnp
from jax import lax
from jax.experimental import pallas as pl
from jax.experimental.pallas import tpu as pltpu
from jax.experimental.pallas import tpu_sc as plsc

N = 10000
D = 128
E = 160000
NC = 2
NS = 16
NW = NC * NS
CHUNK = 128
CPW = -(-E // (NW * CHUNK))
EPAD = NW * CPW * CHUNK
NPAD = 10240
STR = NPAD // NS
SPT = STR // CHUNK
RB = 2000
KB = 2
NB = CPW // KB
F32 = jnp.float32


def _segsum_body(h0, h1, h2, s0, s1, s2, d0, d1, d2, zeros_hbm,
                 o0, o1, o2, src_v, dst_v, rows2, agg, gs, ss):
    hs_refs = (h0, h1, h2)
    src_refs = (s0, s1, s2)
    dst_refs = (d0, d1, d2)
    out_refs = (o0, o1, o2)
    c = lax.axis_index("c")
    s = lax.axis_index("s")
    wid = c * NS + s
    pltpu.sync_copy(zeros_hbm, agg.at[pl.ds(s * STR, STR)])
    plsc.subcore_barrier()
    for et in range(3):
        h = hs_refs[et]
        pltpu.sync_copy(src_refs[et].at[wid], src_v)
        pltpu.sync_copy(dst_refs[et].at[wid], dst_v)

        def slot(k):
            return rows2.at[pl.ds(k * CHUNK, CHUNK)]

        def fire_g(jb):
            def one(k, _):
                pltpu.async_copy(h.at[src_v.at[jb + k]], slot(k), gs)
                return 0
            lax.fori_loop(0, KB, one, 0)

        def drain_g(jb):
            def one(k, _):
                pltpu.make_async_copy(h.at[src_v.at[jb + k]], slot(k),
                                      gs).wait()
                return 0
            lax.fori_loop(0, KB, one, 0)

        def fire_s(jb):
            def one(k, _):
                pltpu.async_copy(slot(k), agg.at[dst_v.at[jb + k]], ss,
                                 add=True)
                return 0
            lax.fori_loop(0, KB, one, 0)

        def drain_s(jb):
            def one(k, _):
                pltpu.make_async_copy(slot(k), agg.at[dst_v.at[jb + k]],
                                      ss).wait()
                return 0
            lax.fori_loop(0, KB, one, 0)

        fire_g(0)

        def body(b, _):
            jb = b * KB
            drain_g(jb)
            fire_s(jb)
            drain_s(jb)

            @pl.when(b + 1 < NB)
            def _():
                fire_g(jb + KB)

            return 0

        lax.fori_loop(0, NB, body, 0)
        plsc.subcore_barrier()
        pltpu.sync_copy(agg.at[pl.ds(s * STR, STR)],
                        out_refs[et].at[pl.ds(c * NPAD + s * STR, STR)])
        if et < 2:
            plsc.subcore_barrier()


@functools.lru_cache(maxsize=None)
def _sc_kernels():
    mesh = plsc.VectorSubcoreMesh(core_axis_name="c", subcore_axis_name="s")
    segsum = pl.kernel(
        _segsum_body,
        out_type=[jax.ShapeDtypeStruct((NC * NPAD, D), F32)
                  for _ in range(3)],
        mesh=mesh,
        scratch_types=[
            pltpu.VMEM((CPW, CHUNK), jnp.int32),
            pltpu.VMEM((CPW, CHUNK), jnp.int32),
            pltpu.VMEM((KB * CHUNK, D), F32),
            pltpu.VMEM_SHARED((NPAD, D), F32),
            pltpu.SemaphoreType.DMA,
            pltpu.SemaphoreType.DMA,
        ],
    )
    return (segsum,)


def _sc_segsum(*args):
    return _sc_kernels()[0](*args)


def _rsq(cnt_ref, k):
    cnt = cnt_ref[k, 0, :, 0] + cnt_ref[k, 1, :, 0]
    return lax.rsqrt(jnp.maximum(cnt, 1.0))


def _prescale_body(x_ref, cnt_ref, a_ref, b_ref, c_ref):
    x = x_ref[...]
    for et, o in enumerate((a_ref, b_ref, c_ref)):
        o[...] = x * _rsq(cnt_ref, et)[:, None]


_tc_prescale = pl.pallas_call(
    _prescale_body,
    grid=(N // RB,),
    in_specs=[
        pl.BlockSpec((RB, D), lambda i: (i, 0)),
        pl.BlockSpec((6, NC, RB, 8), lambda i: (0, 0, i, 0)),
    ],
    out_specs=[pl.BlockSpec((RB, D), lambda i: (i, 0))] * 3,
    out_shape=[jax.ShapeDtypeStruct((N, D), F32)] * 3,
)


def _dense_a_body(s0, s1, s2, cnt_ref, w_ref, bsum_ref, fcw_ref, fcb_ref,
                  hr_ref, sum_ref, sq_ref):
    i = pl.program_id(0)
    acc = jnp.broadcast_to(bsum_ref[...], (RB, D))
    prev = 0.0
    for et, sref in enumerate((s0, s1, s2)):
        cum = sref[0] + sref[1]
        se = (cum - prev) * _rsq(cnt_ref, 3 + et)[:, None]
        prev = cum
        acc = acc + jnp.dot(se, w_ref[et], preferred_element_type=F32,
                            precision=lax.Precision.HIGHEST)
    pre = jnp.dot(acc, fcw_ref[...], preferred_element_type=F32,
                  precision=lax.Precision.HIGHEST) + fcb_ref[...]
    hr = jnp.maximum(pre, 0.0)
    hr_ref[...] = hr

    @pl.when(i == 0)
    def _():
        sum_ref[...] = jnp.zeros_like(sum_ref)
        sq_ref[...] = jnp.zeros_like(sq_ref)

    sum_ref[...] += jnp.broadcast_to(jnp.sum(hr, 0, keepdims=True), (8, D))
    sq_ref[...] += jnp.broadcast_to(jnp.sum(hr * hr, 0, keepdims=True), (8, D))


_tc_dense_a = pl.pallas_call(
    _dense_a_body,
    grid=(N // RB,),
    in_specs=[pl.BlockSpec((NC, RB, D), lambda i: (0, i, 0))] * 3 + [
        pl.BlockSpec((6, NC, RB, 8), lambda i: (0, 0, i, 0)),
        pl.BlockSpec((3, D, D), lambda i: (0, 0, 0)),
        pl.BlockSpec((1, D), lambda i: (0, 0)),
        pl.BlockSpec((D, D), lambda i: (0, 0)),
        pl.BlockSpec((1, D), lambda i: (0, 0)),
    ],
    out_specs=[
        pl.BlockSpec((RB, D), lambda i: (i, 0)),
        pl.BlockSpec((8, D), lambda i: (0, 0)),
        pl.BlockSpec((8, D), lambda i: (0, 0)),
    ],
    out_shape=[
        jax.ShapeDtypeStruct((N, D), F32),
        jax.ShapeDtypeStruct((8, D), F32),
        jax.ShapeDtypeStruct((8, D), F32),
    ],
)


def _make_dense_b(with_hs):
    def body(hr_ref, sum_ref, sq_ref, g_ref, b_ref, cnt_ref, h_ref, *hs_refs):
        mean = sum_ref[0:1, :] * (1.0 / N)
        var = sq_ref[0:1, :] * (1.0 / N) - mean * mean
        inv = lax.rsqrt(var + 1e-5)
        hn = (hr_ref[...] - mean) * (inv * g_ref[...]) + b_ref[...]
        h_ref[...] = hn
        if with_hs:
            for et, o in enumerate(hs_refs):
                o[...] = hn * _rsq(cnt_ref, et)[:, None]

    n_out = 4 if with_hs else 1
    return pl.pallas_call(
        body,
        grid=(N // RB,),
        in_specs=[
            pl.BlockSpec((RB, D), lambda i: (i, 0)),
            pl.BlockSpec((8, D), lambda i: (0, 0)),
            pl.BlockSpec((8, D), lambda i: (0, 0)),
            pl.BlockSpec((1, D), lambda i: (0, 0)),
            pl.BlockSpec((1, D), lambda i: (0, 0)),
            pl.BlockSpec((6, NC, RB, 8), lambda i: (0, 0, i, 0)),
        ],
        out_specs=[pl.BlockSpec((RB, D), lambda i: (i, 0))] * n_out,
        out_shape=[jax.ShapeDtypeStruct((N, D), F32)] * n_out,
    )


_tc_dense_b_hs = _make_dense_b(True)
_tc_dense_b_last = _make_dense_b(False)


def _pad_idx(a, fill):
    a = a.astype(jnp.int32)
    a = jnp.concatenate([a, jnp.full((EPAD - E,), fill, jnp.int32)])
    return a.reshape(NW, CPW, CHUNK)


def kernel(x, params, edge_index_residue, edge_index_seq, edge_index_knn):
    x = x.astype(F32)
    eis = (edge_index_residue, edge_index_seq, edge_index_knn)
    src0 = [_pad_idx(ei[0], 0) for ei in eis]
    srcN = [_pad_idx(ei[0], N) for ei in eis]
    dst0 = [_pad_idx(ei[1], 0) for ei in eis]
    dstN = [_pad_idx(ei[1], N) for ei in eis]

    zerosD = jnp.zeros((STR, D), F32)
    ones_tbl = jnp.ones((N, D), F32)

    din3 = _sc_segsum(ones_tbl, ones_tbl, ones_tbl,
                      src0[0], src0[1], src0[2],
                      dstN[0], dstN[1], dstN[2], zerosD)
    ones_tbl2 = ones_tbl + din3[0][:N] * 0.0
    dout3 = _sc_segsum(ones_tbl2, ones_tbl2, ones_tbl2,
                       dst0[0], dst0[1], dst0[2],
                       srcN[0], srcN[1], srcN[2], zerosD)
    def _decum(outs):
        cums = [o.reshape(NC, NPAD, D)[:, :, :8] for o in outs]
        return [cums[0], cums[1] - cums[0], cums[2] - cums[1]]

    cnts = jnp.stack(_decum(dout3) + _decum(din3))

    hs = _tc_prescale(x, cnts)
    h = None
    for l, p in enumerate(params):
        svals = _sc_segsum(hs[0], hs[1], hs[2],
                           src0[0], src0[1], src0[2],
                           dstN[0], dstN[1], dstN[2], zerosD)
        s3 = [o.reshape(NC, NPAD, D) for o in svals]
        w3 = jnp.stack([p['W_residue'], p['W_seq'], p['W_knn']])
        bsum = (p['b_residue'] + p['b_seq'] + p['b_knn']).reshape(1, D)
        hr, sums, sumsq = _tc_dense_a(s3[0], s3[1], s3[2], cnts, w3, bsum,
                                      p['fcW'], p['fcb'].reshape(1, D))
        g = p['gamma'].reshape(1, D)
        b = p['beta'].reshape(1, D)
        if l + 1 < len(params):
            h, *hs = _tc_dense_b_hs(hr, sums, sumsq, g, b, cnts)
        else:
            (h,) = _tc_dense_b_last(hr, sums, sumsq, g, b, cnts)
    return h


# speedup vs baseline: 1.1191x; 1.1191x over previous
import functools

import jax
import jax.numpy as jnp
from jax import lax
from jax.experimental import pallas as pl
from jax.experimental.pallas import tpu as pltpu
from jax.experimental.pallas import tpu_sc as plsc

N = 10000
D = 128
E = 160000
NC = 2
NS = 16
NW = NC * NS
CHUNK = 128
CPW = -(-E // (NW * CHUNK))
EPAD = NW * CPW * CHUNK
CPW0 = 60
CPW1 = 2 * CPW - CPW0
EF = NS * CPW0 * CHUNK
NPAD = 10240
STR = NPAD // NS
SPT = STR // CHUNK
RB = 2000
KB = 2
NB = CPW // KB
F32 = jnp.float32


def _segsum_body(h0, h1, h2, sf0, sf1, sf2, ss0, ss1, ss2,
                 df0, df1, df2, ds0, ds1, ds2, zeros_hbm,
                 o0, o1, o2, src_v, dst_v, rows2, agg, gs, ss):
    hs_refs = (h0, h1, h2)
    srcf_refs = (sf0, sf1, sf2)
    srcs_refs = (ss0, ss1, ss2)
    dstf_refs = (df0, df1, df2)
    dsts_refs = (ds0, ds1, ds2)
    out_refs = (o0, o1, o2)
    c = lax.axis_index("c")
    s = lax.axis_index("s")
    nb = jnp.where(c == 0, CPW0 // KB, CPW1 // KB)
    pltpu.sync_copy(zeros_hbm, agg.at[pl.ds(s * STR, STR)])
    plsc.subcore_barrier()
    for et in range(3):
        h = hs_refs[et]

        @pl.when(c == 0)
        def _(et=et):
            pltpu.sync_copy(srcf_refs[et].at[s], src_v)
            pltpu.sync_copy(dstf_refs[et].at[s], dst_v)

        @pl.when(c == 1)
        def _(et=et):
            pltpu.sync_copy(srcs_refs[et].at[s], src_v.at[pl.ds(0, CPW1)])
            pltpu.sync_copy(dsts_refs[et].at[s], dst_v.at[pl.ds(0, CPW1)])

        def slot(k):
            return rows2.at[pl.ds(k * CHUNK, CHUNK)]

        def fire_g(jb):
            def one(k, _):
                pltpu.async_copy(h.at[src_v.at[jb + k]], slot(k), gs)
                return 0
            lax.fori_loop(0, KB, one, 0)

        def drain_g(jb):
            def one(k, _):
                pltpu.make_async_copy(h.at[src_v.at[jb + k]], slot(k),
                                      gs).wait()
                return 0
            lax.fori_loop(0, KB, one, 0)

        def fire_s(jb):
            def one(k, _):
                pltpu.async_copy(slot(k), agg.at[dst_v.at[jb + k]], ss,
                                 add=True)
                return 0
            lax.fori_loop(0, KB, one, 0)

        def drain_s(jb):
            def one(k, _):
                pltpu.make_async_copy(slot(k), agg.at[dst_v.at[jb + k]],
                                      ss).wait()
                return 0
            lax.fori_loop(0, KB, one, 0)

        fire_g(0)

        def body(b, _):
            jb = b * KB
            drain_g(jb)
            fire_s(jb)
            drain_s(jb)

            @pl.when(b + 1 < nb)
            def _():
                fire_g(jb + KB)

            return 0

        lax.fori_loop(0, nb, body, 0)
        plsc.subcore_barrier()
        pltpu.sync_copy(agg.at[pl.ds(s * STR, STR)],
                        out_refs[et].at[pl.ds(c * NPAD + s * STR, STR)])
        if et < 2:
            plsc.subcore_barrier()


@functools.lru_cache(maxsize=None)
def _sc_kernels():
    mesh = plsc.VectorSubcoreMesh(core_axis_name="c", subcore_axis_name="s")
    segsum = pl.kernel(
        _segsum_body,
        out_type=[jax.ShapeDtypeStruct((NC * NPAD, D), F32)
                  for _ in range(3)],
        mesh=mesh,
        scratch_types=[
            pltpu.VMEM((CPW0, CHUNK), jnp.int32),
            pltpu.VMEM((CPW0, CHUNK), jnp.int32),
            pltpu.VMEM((KB * CHUNK, D), F32),
            pltpu.VMEM_SHARED((NPAD, D), F32),
            pltpu.SemaphoreType.DMA,
            pltpu.SemaphoreType.DMA,
        ],
    )
    return (segsum,)


def _sc_segsum(*args):
    return _sc_kernels()[0](*args)


def _rsq(cnt_ref, k):
    cnt = cnt_ref[k, 0, :, 0] + cnt_ref[k, 1, :, 0]
    return lax.rsqrt(jnp.maximum(cnt, 1.0))


def _prescale_body(x_ref, cnt_ref, a_ref, b_ref, c_ref):
    x = x_ref[...]
    for et, o in enumerate((a_ref, b_ref, c_ref)):
        o[...] = x * _rsq(cnt_ref, et)[:, None]


_tc_prescale = pl.pallas_call(
    _prescale_body,
    grid=(N // RB,),
    in_specs=[
        pl.BlockSpec((RB, D), lambda i: (i, 0)),
        pl.BlockSpec((6, NC, RB, 8), lambda i: (0, 0, i, 0)),
    ],
    out_specs=[pl.BlockSpec((RB, D), lambda i: (i, 0))] * 3,
    out_shape=[jax.ShapeDtypeStruct((N, D), F32)] * 3,
)


def _dense_a_body(s0, s1, s2, cnt_ref, w_ref, bsum_ref, fcw_ref, fcb_ref,
                  hr_ref, sum_ref, sq_ref):
    i = pl.program_id(0)
    acc = jnp.broadcast_to(bsum_ref[...], (RB, D))
    prev = 0.0
    for et, sref in enumerate((s0, s1, s2)):
        cum = sref[0] + sref[1]
        se = (cum - prev) * _rsq(cnt_ref, 3 + et)[:, None]
        prev = cum
        acc = acc + jnp.dot(se, w_ref[et], preferred_element_type=F32,
                            precision=lax.Precision.HIGHEST)
    pre = jnp.dot(acc, fcw_ref[...], preferred_element_type=F32,
                  precision=lax.Precision.HIGHEST) + fcb_ref[...]
    hr = jnp.maximum(pre, 0.0)
    hr_ref[...] = hr

    @pl.when(i == 0)
    def _():
        sum_ref[...] = jnp.zeros_like(sum_ref)
        sq_ref[...] = jnp.zeros_like(sq_ref)

    sum_ref[...] += jnp.broadcast_to(jnp.sum(hr, 0, keepdims=True), (8, D))
    sq_ref[...] += jnp.broadcast_to(jnp.sum(hr * hr, 0, keepdims=True), (8, D))


_tc_dense_a = pl.pallas_call(
    _dense_a_body,
    grid=(N // RB,),
    in_specs=[pl.BlockSpec((NC, RB, D), lambda i: (0, i, 0))] * 3 + [
        pl.BlockSpec((6, NC, RB, 8), lambda i: (0, 0, i, 0)),
        pl.BlockSpec((3, D, D), lambda i: (0, 0, 0)),
        pl.BlockSpec((1, D), lambda i: (0, 0)),
        pl.BlockSpec((D, D), lambda i: (0, 0)),
        pl.BlockSpec((1, D), lambda i: (0, 0)),
    ],
    out_specs=[
        pl.BlockSpec((RB, D), lambda i: (i, 0)),
        pl.BlockSpec((8, D), lambda i: (0, 0)),
        pl.BlockSpec((8, D), lambda i: (0, 0)),
    ],
    out_shape=[
        jax.ShapeDtypeStruct((N, D), F32),
        jax.ShapeDtypeStruct((8, D), F32),
        jax.ShapeDtypeStruct((8, D), F32),
    ],
)


def _make_dense_b(with_hs):
    def body(hr_ref, sum_ref, sq_ref, g_ref, b_ref, cnt_ref, h_ref, *hs_refs):
        mean = sum_ref[0:1, :] * (1.0 / N)
        var = sq_ref[0:1, :] * (1.0 / N) - mean * mean
        inv = lax.rsqrt(var + 1e-5)
        hn = (hr_ref[...] - mean) * (inv * g_ref[...]) + b_ref[...]
        h_ref[...] = hn
        if with_hs:
            for et, o in enumerate(hs_refs):
                o[...] = hn * _rsq(cnt_ref, et)[:, None]

    n_out = 4 if with_hs else 1
    return pl.pallas_call(
        body,
        grid=(N // RB,),
        in_specs=[
            pl.BlockSpec((RB, D), lambda i: (i, 0)),
            pl.BlockSpec((8, D), lambda i: (0, 0)),
            pl.BlockSpec((8, D), lambda i: (0, 0)),
            pl.BlockSpec((1, D), lambda i: (0, 0)),
            pl.BlockSpec((1, D), lambda i: (0, 0)),
            pl.BlockSpec((6, NC, RB, 8), lambda i: (0, 0, i, 0)),
        ],
        out_specs=[pl.BlockSpec((RB, D), lambda i: (i, 0))] * n_out,
        out_shape=[jax.ShapeDtypeStruct((N, D), F32)] * n_out,
    )


_tc_dense_b_hs = _make_dense_b(True)
_tc_dense_b_last = _make_dense_b(False)


def _pad_idx(a, fill):
    a = a.astype(jnp.int32)
    a = jnp.concatenate([a, jnp.full((EPAD - E,), fill, jnp.int32)])
    return (a[:EF].reshape(NS, CPW0, CHUNK),
            a[EF:].reshape(NS, CPW1, CHUNK))


def kernel(x, params, edge_index_residue, edge_index_seq, edge_index_knn):
    x = x.astype(F32)
    eis = (edge_index_residue, edge_index_seq, edge_index_knn)
    src0 = [_pad_idx(ei[0], 0) for ei in eis]
    srcN = [_pad_idx(ei[0], N) for ei in eis]
    dst0 = [_pad_idx(ei[1], 0) for ei in eis]
    dstN = [_pad_idx(ei[1], N) for ei in eis]

    def _idx12(gather, scatter):
        return ([g[0] for g in gather] + [g[1] for g in gather]
                + [sc[0] for sc in scatter] + [sc[1] for sc in scatter])

    zerosD = jnp.zeros((STR, D), F32)
    ones_tbl = jnp.ones((N, D), F32)

    din3 = _sc_segsum(ones_tbl, ones_tbl, ones_tbl,
                      *_idx12(src0, dstN), zerosD)
    ones_tbl2 = ones_tbl + din3[0][:N] * 0.0
    dout3 = _sc_segsum(ones_tbl2, ones_tbl2, ones_tbl2,
                       *_idx12(dst0, srcN), zerosD)
    def _decum(outs):
        cums = [o.reshape(NC, NPAD, D)[:, :, :8] for o in outs]
        return [cums[0], cums[1] - cums[0], cums[2] - cums[1]]

    cnts = jnp.stack(_decum(dout3) + _decum(din3))

    hs = _tc_prescale(x, cnts)
    h = None
    for l, p in enumerate(params):
        svals = _sc_segsum(hs[0], hs[1], hs[2],
                           *_idx12(src0, dstN), zerosD)
        s3 = [o.reshape(NC, NPAD, D) for o in svals]
        w3 = jnp.stack([p['W_residue'], p['W_seq'], p['W_knn']])
        bsum = (p['b_residue'] + p['b_seq'] + p['b_knn']).reshape(1, D)
        hr, sums, sumsq = _tc_dense_a(s3[0], s3[1], s3[2], cnts, w3, bsum,
                                      p['fcW'], p['fcb'].reshape(1, D))
        g = p['gamma'].reshape(1, D)
        b = p['beta'].reshape(1, D)
        if l + 1 < len(params):
            h, *hs = _tc_dense_b_hs(hr, sums, sumsq, g, b, cnts)
        else:
            (h,) = _tc_dense_b_last(hr, sums, sumsq, g, b, cnts)
    return h
